# baseline (speedup 1.0000x reference)
import functools

import jax
import jax.numpy as jnp
from jax import lax
from jax.experimental import pallas as pl
from jax.experimental.pallas import tpu as pltpu
from jax.experimental.pallas import tpu_sc as plsc

N_CAT = 26
N_CONT = 13
VOCAB = 100000
DIM = 16
BATCH = 16384

NUM_CORES = 2
NUM_SUBCORES = 16
NW = NUM_CORES * NUM_SUBCORES
RPW = BATCH // NW
GRP = RPW // 16
CONT_CHUNK = 128


def _body(x_hbm, tab_hbm, bias_hbm, cw_hbm, cb_hbm,
          cat_out, cont_out,
          xv, idxv, maskv, embv, biasv, cwv, cbv, contv,
          sem_g):
    wid = lax.axis_index("s") * NUM_CORES + lax.axis_index("c")
    b0 = wid * RPW

    pltpu.sync_copy(x_hbm.at[pl.ds(b0, RPW)], xv)
    pltpu.sync_copy(bias_hbm, biasv)
    pltpu.sync_copy(cw_hbm, cwv)
    pltpu.sync_copy(cb_hbm, cbv)

    lanes = lax.iota(jnp.int32, 16)

    for i in range(N_CAT):
        col = jnp.full((16,), i, jnp.int32)

        def build(g, c, col=col):
            rows = g * 16 + lanes
            vals = plsc.load_gather(xv, [rows, col])
            ivals = vals.astype(jnp.int32)
            idxv[pl.ds(g * 16, 16)] = ivals + i * (VOCAB + 1)
            maskv[pl.ds(g * 16, 16)] = jnp.where(
                ivals == 0, jnp.zeros((16,), jnp.float32),
                jnp.ones((16,), jnp.float32))
            return c

        lax.fori_loop(0, GRP, build, 0)

        pltpu.async_copy(tab_hbm.at[idxv], embv, sem_g).wait()

        bias_i = biasv[i]

        def post(r, c, bias_i=bias_i):
            m = plsc.load_gather(maskv, [jnp.full((16,), r, jnp.int32)])
            embv[r] = embv[r] * m + bias_i
            return c

        lax.fori_loop(0, RPW, post, 0)

        pltpu.sync_copy(embv, cat_out.at[pl.ds(b0, RPW), pl.ds(i * DIM, DIM)])

    wvecs = [cwv[j] for j in range(N_CONT)]
    bvecs = [cbv[j] for j in range(N_CONT)]
    for c in range(RPW // CONT_CHUNK):
        def cont_row(r, carry):
            row = c * CONT_CHUNK + r
            rowv = jnp.full((16,), row, jnp.int32)
            for j in range(N_CONT):
                s = plsc.load_gather(
                    xv, [rowv, jnp.full((16,), N_CAT + j, jnp.int32)])
                contv[r, pl.ds(j * DIM, DIM)] = s * wvecs[j] + bvecs[j]
            return carry

        lax.fori_loop(0, CONT_CHUNK, cont_row, 0)
        pltpu.sync_copy(
            contv, cont_out.at[pl.ds(b0 + c * CONT_CHUNK, CONT_CHUNK)])


@jax.jit
def kernel(X, cat_tables, cat_biases, cont_weight, cont_bias):
    tab_flat = cat_tables.reshape(N_CAT * (VOCAB + 1), DIM)
    mesh = plsc.VectorSubcoreMesh(
        core_axis_name="c", subcore_axis_name="s",
        num_cores=NUM_CORES, num_subcores=NUM_SUBCORES)
    run = pl.kernel(
        _body,
        out_type=(
            jax.ShapeDtypeStruct((BATCH, N_CAT * DIM), jnp.float32),
            jax.ShapeDtypeStruct((BATCH, N_CONT * DIM), jnp.float32),
        ),
        mesh=mesh,
        compiler_params=pltpu.CompilerParams(
            use_tc_tiling_on_sc=False, needs_layout_passes=False),
        scratch_types=[
            pltpu.VMEM((RPW, N_CAT + N_CONT), jnp.float32),
            pltpu.VMEM((RPW,), jnp.int32),
            pltpu.VMEM((RPW,), jnp.float32),
            pltpu.VMEM((RPW, DIM), jnp.float32),
            pltpu.VMEM((N_CAT, DIM), jnp.float32),
            pltpu.VMEM((N_CONT, DIM), jnp.float32),
            pltpu.VMEM((N_CONT, DIM), jnp.float32),
            pltpu.VMEM((CONT_CHUNK, N_CONT * DIM), jnp.float32),
            pltpu.SemaphoreType.DMA,
        ],
    )
    x_cat, x_cont = run(X, tab_flat, cat_biases, cont_weight, cont_bias)
    return (x_cat, x_cont)
